# baseline (speedup 1.0000x reference)
import functools

import jax
import jax.numpy as jnp
from jax import lax
from jax.experimental import pallas as pl
from jax.experimental.pallas import tpu as pltpu
from jax.experimental.pallas import tpu_sc as plsc

_CH = 128


def _mesh():
    return plsc.VectorSubcoreMesh(core_axis_name="c", subcore_axis_name="s")


def _gather_kernel(NW, NC, n_ch, b_per_w, D):
    @functools.partial(
        pl.kernel,
        out_type=jax.ShapeDtypeStruct((NW, b_per_w, D), jnp.float32),
        mesh=_mesh(),
        compiler_params=pltpu.CompilerParams(
            needs_layout_passes=False, use_tc_tiling_on_sc=False),
        scratch_types=[
            pltpu.VMEM((n_ch, _CH), jnp.int32),
            pltpu.VMEM((2, _CH, D), jnp.float32),
            pltpu.SemaphoreType.DMA,
            pltpu.SemaphoreType.DMA,
        ],
    )
    def k1(idx_hbm, tab_hbm, out_hbm, idx_v, buf, sem, osem):
        wid = lax.axis_index("s") * NC + lax.axis_index("c")
        pltpu.sync_copy(idx_hbm.at[wid], idx_v)

        def fire(j):
            return pltpu.async_copy(
                tab_hbm.at[idx_v.at[j]], buf.at[j % 2], sem)

        outs = [None, None]
        pending = fire(0)
        for j in range(n_ch):
            pending.wait()
            if j + 1 < n_ch:
                if outs[(j + 1) % 2] is not None:
                    outs[(j + 1) % 2].wait()
                pending = fire(j + 1)
            outs[j % 2] = pltpu.async_copy(
                buf.at[j % 2],
                out_hbm.at[wid].at[pl.ds(j * _CH, _CH)], osem)
        for c in outs:
            if c is not None:
                c.wait()

    return k1


def _dot_kernel(NW, NC, n_ch, b_per_w, D, L):
    @functools.partial(
        pl.kernel,
        out_type=jax.ShapeDtypeStruct((NW, b_per_w), jnp.float32),
        mesh=_mesh(),
        compiler_params=pltpu.CompilerParams(
            needs_layout_passes=False, use_tc_tiling_on_sc=False),
        scratch_types=[
            pltpu.VMEM((n_ch, _CH), jnp.int32),
            pltpu.VMEM((2, _CH, D), jnp.float32),
            pltpu.VMEM((2, _CH, D), jnp.float32),
            pltpu.VMEM((b_per_w,), jnp.float32),
            pltpu.SemaphoreType.DMA,
        ],
    )
    def k2(idx_hbm, tab_hbm, urows_hbm, out_hbm,
           idx_v, vbuf, ubuf, outv, sem):
        wid = lax.axis_index("s") * NC + lax.axis_index("c")
        pltpu.sync_copy(idx_hbm.at[wid], idx_v)

        iota = lax.iota(jnp.int32, L)

        def fire(j):
            s = j % 2
            return (
                pltpu.async_copy(tab_hbm.at[idx_v.at[j]], vbuf.at[s], sem),
                pltpu.async_copy(
                    urows_hbm.at[wid].at[pl.ds(j * _CH, _CH)],
                    ubuf.at[s], sem),
            )

        pending = fire(0)
        for j in range(n_ch):
            for c in pending:
                c.wait()
            if j + 1 < n_ch:
                nxt = fire(j + 1)
            s = j % 2
            urows = ubuf.at[s]
            vrows = vbuf.at[s]

            @plsc.parallel_loop(0, _CH // L, step=1, unroll=2)
            def group_body(g):
                rows = g * L + iota
                accs = [jnp.zeros((L,), jnp.float32) for _ in range(4)]
                for d in range(D):
                    dv = jnp.full((L,), d, jnp.int32)
                    u = plsc.load_gather(urows, [rows, dv])
                    v = plsc.load_gather(vrows, [rows, dv])
                    accs[d % 4] = accs[d % 4] + u * v
                outv[pl.ds(j * _CH + g * L, L)] = (
                    (accs[0] + accs[1]) + (accs[2] + accs[3]))
            if j + 1 < n_ch:
                pending = nxt

        pltpu.sync_copy(outv, out_hbm.at[wid])

    return k2


def kernel(x, W, H):
    B = x.shape[0]
    D = W.shape[1]
    info = plsc.get_sparse_core_info()
    NC, NS, L = info.num_cores, info.num_subcores, info.num_lanes
    NW = NC * NS
    b_per_w = B // NW
    n_ch = b_per_w // _CH

    ug = x[:, 0].reshape(NW, n_ch, _CH)
    vg = x[:, 1].reshape(NW, n_ch, _CH)

    urows = _gather_kernel(NW, NC, n_ch, b_per_w, D)(ug, W)
    out = _dot_kernel(NW, NC, n_ch, b_per_w, D, L)(vg, H, urows)
    return out.reshape(B)


# speedup vs baseline: 1.0013x; 1.0013x over previous
import functools

import jax
import jax.numpy as jnp
from jax import lax
from jax.experimental import pallas as pl
from jax.experimental.pallas import tpu as pltpu
from jax.experimental.pallas import tpu_sc as plsc


def kernel(x, W, H):
    B = x.shape[0]
    D = W.shape[1]
    info = plsc.get_sparse_core_info()
    NC, NS, L = info.num_cores, info.num_subcores, info.num_lanes
    NW = NC * NS
    b_per_w = B // NW
    CH = 128
    n_ch = b_per_w // CH

    ug = x[:, 0].reshape(NW, n_ch, CH)
    vg = x[:, 1].reshape(NW, n_ch, CH)

    mesh = plsc.VectorSubcoreMesh(core_axis_name="c", subcore_axis_name="s")

    @functools.partial(
        pl.kernel,
        out_type=jax.ShapeDtypeStruct((NW, b_per_w), jnp.float32),
        mesh=mesh,
        compiler_params=pltpu.CompilerParams(
            needs_layout_passes=False, use_tc_tiling_on_sc=False),
        scratch_types=[
            pltpu.VMEM((n_ch, CH), jnp.int32),
            pltpu.VMEM((n_ch, CH), jnp.int32),
            pltpu.VMEM((2, CH, D), jnp.float32),
            pltpu.VMEM((2, CH, D), jnp.float32),
            pltpu.VMEM((b_per_w,), jnp.float32),
            pltpu.SemaphoreType.DMA,
        ],
    )
    def sc_kernel(ug_hbm, vg_hbm, w_hbm, h_hbm, out_hbm,
                  ug_v, vg_v, ubuf, vbuf, outv, sem):
        wid = lax.axis_index("s") * NC + lax.axis_index("c")

        idx_copies = [
            pltpu.async_copy(ug_hbm.at[wid], ug_v, sem),
            pltpu.async_copy(vg_hbm.at[wid], vg_v, sem),
        ]
        for c in idx_copies:
            c.wait()

        iota = lax.iota(jnp.int32, L)

        def fire(j):
            s = j % 2
            return (pltpu.async_copy(w_hbm.at[ug_v.at[j]], ubuf.at[s], sem),
                    pltpu.async_copy(h_hbm.at[vg_v.at[j]], vbuf.at[s], sem))

        pending = fire(0)
        for j in range(n_ch):
            for c in pending:
                c.wait()
            if j + 1 < n_ch:
                nxt = fire(j + 1)
            s = j % 2
            urows = ubuf.at[s]
            vrows = vbuf.at[s]

            @plsc.parallel_loop(0, CH // L, step=1, unroll=2)
            def group_body(g):
                rows = g * L + iota
                accs = [jnp.zeros((L,), jnp.float32) for _ in range(4)]
                for d in range(D):
                    dv = jnp.full((L,), d, jnp.int32)
                    u = plsc.load_gather(urows, [rows, dv])
                    v = plsc.load_gather(vrows, [rows, dv])
                    accs[d % 4] = accs[d % 4] + u * v
                outv[pl.ds(j * CH + g * L, L)] = (
                    (accs[0] + accs[1]) + (accs[2] + accs[3]))
            if j + 1 < n_ch:
                pending = nxt

        pltpu.sync_copy(outv, out_hbm.at[wid])

    out = sc_kernel(ug, vg, W, H)
    return out.reshape(B)


# speedup vs baseline: 1.0101x; 1.0087x over previous
import functools

import jax
import jax.numpy as jnp
from jax import lax
from jax.experimental import pallas as pl
from jax.experimental.pallas import tpu as pltpu
from jax.experimental.pallas import tpu_sc as plsc


def kernel(x, W, H):
    B = x.shape[0]
    D = W.shape[1]
    info = plsc.get_sparse_core_info()
    NC, NS, L = info.num_cores, info.num_subcores, info.num_lanes
    NW = NC * NS
    b_per_w = B // NW
    CH = 128
    n_ch = b_per_w // CH

    Wv = W.reshape(W.shape[0] // 2, 2 * D)
    Hv = H.reshape(H.shape[0] // 2, 2 * D)

    u_idx = x[:, 0]
    v_idx = x[:, 1]
    ug = (u_idx // 2).reshape(NW, n_ch, CH)
    vg = (v_idx // 2).reshape(NW, n_ch, CH)
    uc = ((u_idx % 2) * D).reshape(NW, b_per_w)
    vc = ((v_idx % 2) * D).reshape(NW, b_per_w)

    mesh = plsc.VectorSubcoreMesh(core_axis_name="c", subcore_axis_name="s")

    @functools.partial(
        pl.kernel,
        out_type=jax.ShapeDtypeStruct((NW, b_per_w), jnp.float32),
        mesh=mesh,
        compiler_params=pltpu.CompilerParams(
            needs_layout_passes=False, use_tc_tiling_on_sc=True),
        scratch_types=[
            pltpu.VMEM((n_ch, CH), jnp.int32),
            pltpu.VMEM((n_ch, CH), jnp.int32),
            pltpu.VMEM((b_per_w,), jnp.int32),
            pltpu.VMEM((b_per_w,), jnp.int32),
            pltpu.VMEM((2, CH, 2 * D), jnp.float32),
            pltpu.VMEM((2, CH, 2 * D), jnp.float32),
            pltpu.VMEM((b_per_w,), jnp.float32),
            pltpu.SemaphoreType.DMA,
        ],
    )
    def sc_kernel(ug_hbm, vg_hbm, uc_hbm, vc_hbm, w_hbm, h_hbm, out_hbm,
                  ug_v, vg_v, uc_v, vc_v, ubuf, vbuf, outv, sem):
        wid = lax.axis_index("s") * NC + lax.axis_index("c")

        idx_copies = [
            pltpu.async_copy(ug_hbm.at[wid], ug_v, sem),
            pltpu.async_copy(vg_hbm.at[wid], vg_v, sem),
            pltpu.async_copy(uc_hbm.at[wid], uc_v, sem),
            pltpu.async_copy(vc_hbm.at[wid], vc_v, sem),
        ]
        for c in idx_copies:
            c.wait()

        iota = lax.iota(jnp.int32, L)

        def fire(j):
            s = j % 2
            return (pltpu.async_copy(w_hbm.at[ug_v.at[j]], ubuf.at[s], sem),
                    pltpu.async_copy(h_hbm.at[vg_v.at[j]], vbuf.at[s], sem))

        pending = fire(0)
        for j in range(n_ch):
            for c in pending:
                c.wait()
            if j + 1 < n_ch:
                nxt = fire(j + 1)
            s = j % 2
            urows = ubuf.at[s]
            vrows = vbuf.at[s]

            @plsc.parallel_loop(0, CH // L, step=1, unroll=2)
            def group_body(g):
                rows = g * L + iota
                ubase = uc_v[pl.ds(j * CH + g * L, L)]
                vbase = vc_v[pl.ds(j * CH + g * L, L)]
                accs = [jnp.zeros((L,), jnp.float32) for _ in range(4)]
                for d in range(D):
                    u = plsc.load_gather(urows, [rows, ubase + d])
                    v = plsc.load_gather(vrows, [rows, vbase + d])
                    accs[d % 4] = accs[d % 4] + u * v
                outv[pl.ds(j * CH + g * L, L)] = (
                    (accs[0] + accs[1]) + (accs[2] + accs[3]))
            if j + 1 < n_ch:
                pending = nxt

        pltpu.sync_copy(outv, out_hbm.at[wid])

    out = sc_kernel(ug, vg, uc, vc, Wv, Hv)
    return out.reshape(B)
